# baseline (speedup 1.0000x reference)
import functools

import jax
import jax.numpy as jnp
from jax import lax
from jax.experimental import pallas as pl
from jax.experimental.pallas import tpu as pltpu
from jax.experimental.pallas import tpu_sc as plsc

_SCALE = 65536.0
_NSEG = 16
_SLOPE_MASK = -8388608
_B_MASK = 0x1FFFF

_EYTZ_PERM = (8, 4, 9, 2, 10, 5, 11, 1, 12, 6, 13, 3, 14, 7, 15)

_TC_ROWS_OF_16384 = 6144


def _pack_tables(intercepts, signs, exps):
    sign_bit = ((1 - signs) // 2).astype(jnp.int32)
    slope_bits = (sign_bit << 31) | ((127 + exps) << 23)
    return slope_bits | (intercepts + 65536)


def _thresholds(breakpoints):
    bpf = breakpoints.astype(jnp.float32)
    pos = bpf / _SCALE
    neg = jnp.nextafter(bpf - 1.0, jnp.float32(jnp.inf)) / _SCALE
    return jnp.where(breakpoints > 0, pos, neg)


def _eytz_tree(thresholds):
    return jnp.zeros((16,), jnp.float32).at[jnp.array(_EYTZ_PERM)].set(thresholds)


def _tc_body(t_ref, packed_ref, x_ref, o_ref):
    x = x_ref[...]
    acc = jnp.where(x >= t_ref[0], packed_ref[1], packed_ref[0])
    for j in range(1, _NSEG - 1):
        acc = jnp.where(x >= t_ref[j], packed_ref[j + 1], acc)
    slope = lax.bitcast_convert_type(acc & _SLOPE_MASK, jnp.float32)
    b = (acc & _B_MASK).astype(jnp.float32) * (1.0 / _SCALE) - 1.0
    o_ref[...] = b + slope * x


def _tc_call(x2, thresholds, packed):
    rows, cols = x2.shape
    br = min(512, rows)
    grid = rows // br
    return pl.pallas_call(
        _tc_body,
        grid=(grid,),
        in_specs=[
            pl.BlockSpec(memory_space=pltpu.SMEM),
            pl.BlockSpec(memory_space=pltpu.SMEM),
            pl.BlockSpec((br, cols), lambda i: (i, 0)),
        ],
        out_specs=pl.BlockSpec((br, cols), lambda i: (i, 0)),
        out_shape=jax.ShapeDtypeStruct((rows, cols), jnp.float32),
    )(thresholds, packed, x2)


_SC_CH = 16384


def _sc_call(x, tree, ptab):
    n = x.shape[0]
    nw = 32
    per_w = n // nw
    nch = per_w // _SC_CH
    assert per_w % _SC_CH == 0
    mesh = plsc.VectorSubcoreMesh(core_axis_name="c", subcore_axis_name="s")

    @functools.partial(
        pl.kernel,
        out_type=jax.ShapeDtypeStruct((n,), jnp.float32),
        mesh=mesh,
        compiler_params=pltpu.CompilerParams(needs_layout_passes=False),
        scratch_types=[
            pltpu.VMEM((16,), jnp.float32),
            pltpu.VMEM((16,), jnp.int32),
            pltpu.VMEM((_SC_CH,), jnp.float32),
            pltpu.VMEM((_SC_CH,), jnp.float32),
            pltpu.VMEM((_SC_CH,), jnp.float32),
            pltpu.VMEM((_SC_CH,), jnp.float32),
            pltpu.SemaphoreType.DMA,
            pltpu.SemaphoreType.DMA,
            pltpu.SemaphoreType.DMA,
            pltpu.SemaphoreType.DMA,
        ],
    )
    def sck(x_hbm, tree_hbm, ptab_hbm, o_hbm, tree_v, ptab_v,
            in0, in1, out0, out1, si0, si1, so0, so1):
        wid = lax.axis_index("s") * 2 + lax.axis_index("c")
        base = wid * per_w
        pltpu.sync_copy(tree_hbm, tree_v)
        pltpu.sync_copy(ptab_hbm, ptab_v)
        ins = (in0, in1)
        outs = (out0, out1)
        sis = (si0, si1)
        sos = (so0, so1)

        def start_in(g, s):
            pltpu.async_copy(
                x_hbm.at[pl.ds(base + g * _SC_CH, _SC_CH)], ins[s], sis[s])

        def start_out(g, s):
            pltpu.async_copy(
                outs[s], o_hbm.at[pl.ds(base + g * _SC_CH, _SC_CH)], sos[s])

        def wait_in(s):
            pltpu.make_async_copy(
                x_hbm.at[pl.ds(base, _SC_CH)], ins[s], sis[s]).wait()

        def wait_out(s):
            pltpu.make_async_copy(
                outs[s], o_hbm.at[pl.ds(base, _SC_CH)], sos[s]).wait()

        def compute(s):
            xin = ins[s]
            yout = outs[s]

            def vbody(o2):
                xv = xin[pl.ds(o2, 16)]
                node = jnp.ones((16,), jnp.int32)
                for _ in range(4):
                    tv = plsc.load_gather(tree_v, [node])
                    node = node + node + (xv >= tv).astype(jnp.int32)
                pk = plsc.load_gather(ptab_v, [node - 16])
                slope = plsc.bitcast(pk & _SLOPE_MASK, jnp.float32)
                b = (pk & _B_MASK).astype(jnp.float32) * (1.0 / _SCALE) - 1.0
                yout[pl.ds(o2, 16)] = b + slope * xv

            plsc.parallel_loop(0, _SC_CH, 16, unroll=8)(vbody)

        start_in(0, 0)
        start_in(1, 1)
        for g in (0, 1):
            wait_in(g)
            compute(g)
            start_in(g + 2, g)
            start_out(g, g)

        def pair_body(g2, c):
            g0 = 2 + 2 * g2
            for s in (0, 1):
                g = g0 + s
                wait_in(s)
                wait_out(s)
                compute(s)
                start_in(g + 2, s)
                start_out(g, s)
            return c

        lax.fori_loop(0, (nch - 4) // 2, pair_body, 0)

        for g in (nch - 2, nch - 1):
            s = g & 1
            wait_in(s)
            wait_out(s)
            compute(s)
            start_out(g, s)
        wait_out(0)
        wait_out(1)

    return sck(x, tree, ptab)


def kernel(x, breakpoints, intercepts, signs, exps):
    packed = _pack_tables(intercepts, signs, exps)
    thr = _thresholds(breakpoints)
    n = x.shape[0]
    cols = 1024
    k = _TC_ROWS_OF_16384 * (n // 16384)
    parts = []
    if k > 0:
        parts.append(_tc_call(x[:k].reshape(k // cols, cols), thr, packed).reshape(-1))
    if k < n:
        parts.append(_sc_call(x[k:], _eytz_tree(thr), packed))
    out = parts[0] if len(parts) == 1 else jnp.concatenate(parts)
    return out.astype(x.dtype)


# speedup vs baseline: 1.0876x; 1.0876x over previous
import functools

import jax
import jax.numpy as jnp
from jax import lax
from jax.experimental import pallas as pl
from jax.experimental.pallas import tpu as pltpu
from jax.experimental.pallas import tpu_sc as plsc

_SCALE = 65536.0
_NSEG = 16
_SLOPE_MASK = -8388608
_B_MASK = 0x1FFFF

_EYTZ_PERM = (8, 4, 9, 2, 10, 5, 11, 1, 12, 6, 13, 3, 14, 7, 15)

_TC_ROWS_OF_16384 = 6144


def _pack_tables(intercepts, signs, exps):
    sign_bit = ((1 - signs) // 2).astype(jnp.int32)
    slope_bits = (sign_bit << 31) | ((127 + exps) << 23)
    return slope_bits | (intercepts + 65536)


def _thresholds(breakpoints):
    bpf = breakpoints.astype(jnp.float32)
    pos = bpf / _SCALE
    neg = jnp.nextafter(bpf - 1.0, jnp.float32(jnp.inf)) / _SCALE
    return jnp.where(breakpoints > 0, pos, neg)


def _eytz_tree(thresholds):
    return jnp.zeros((16,), jnp.float32).at[jnp.array(_EYTZ_PERM)].set(thresholds)


def _tc_body(t_ref, packed_ref, x_ref, o_ref):
    x = x_ref[...]
    acc = jnp.where(x >= t_ref[0], packed_ref[1], packed_ref[0])
    for j in range(1, _NSEG - 1):
        acc = jnp.where(x >= t_ref[j], packed_ref[j + 1], acc)
    slope = lax.bitcast_convert_type(acc & _SLOPE_MASK, jnp.float32)
    b = (acc & _B_MASK).astype(jnp.float32) * (1.0 / _SCALE) - 1.0
    o_ref[...] = b + slope * x


def _tc_call(x2, thresholds, packed, out_rows=None):
    rows, cols = x2.shape
    out_rows = rows if out_rows is None else out_rows
    br = min(512, out_rows)
    grid = out_rows // br
    return pl.pallas_call(
        _tc_body,
        grid=(grid,),
        in_specs=[
            pl.BlockSpec(memory_space=pltpu.SMEM),
            pl.BlockSpec(memory_space=pltpu.SMEM),
            pl.BlockSpec((br, cols), lambda i: (i, 0)),
        ],
        out_specs=pl.BlockSpec((br, cols), lambda i: (i, 0)),
        out_shape=jax.ShapeDtypeStruct((out_rows, cols), jnp.float32),
    )(thresholds, packed, x2)


_SC_CH = 16384


def _sc_call(x, tree, ptab, skip=0):
    n = x.shape[0]
    nw = 32
    per_w = (n - skip) // nw
    nch = per_w // _SC_CH
    assert (n - skip) % nw == 0 and per_w % _SC_CH == 0 and nch % 2 == 0
    mesh = plsc.VectorSubcoreMesh(core_axis_name="c", subcore_axis_name="s")

    @functools.partial(
        pl.kernel,
        out_type=jax.ShapeDtypeStruct((n,), jnp.float32),
        mesh=mesh,
        compiler_params=pltpu.CompilerParams(needs_layout_passes=False),
        scratch_types=[
            pltpu.VMEM((16,), jnp.float32),
            pltpu.VMEM((16,), jnp.int32),
            pltpu.VMEM((_SC_CH,), jnp.float32),
            pltpu.VMEM((_SC_CH,), jnp.float32),
            pltpu.VMEM((_SC_CH,), jnp.float32),
            pltpu.VMEM((_SC_CH,), jnp.float32),
            pltpu.SemaphoreType.DMA,
            pltpu.SemaphoreType.DMA,
            pltpu.SemaphoreType.DMA,
            pltpu.SemaphoreType.DMA,
        ],
    )
    def sck(x_hbm, tree_hbm, ptab_hbm, o_hbm, tree_v, ptab_v,
            in0, in1, out0, out1, si0, si1, so0, so1):
        wid = lax.axis_index("s") * 2 + lax.axis_index("c")
        base = skip + wid * per_w
        pltpu.sync_copy(tree_hbm, tree_v)
        pltpu.sync_copy(ptab_hbm, ptab_v)
        ins = (in0, in1)
        outs = (out0, out1)
        sis = (si0, si1)
        sos = (so0, so1)

        def start_in(g, s):
            pltpu.async_copy(
                x_hbm.at[pl.ds(base + g * _SC_CH, _SC_CH)], ins[s], sis[s])

        def start_out(g, s):
            pltpu.async_copy(
                outs[s], o_hbm.at[pl.ds(base + g * _SC_CH, _SC_CH)], sos[s])

        def wait_in(s):
            pltpu.make_async_copy(
                x_hbm.at[pl.ds(base, _SC_CH)], ins[s], sis[s]).wait()

        def wait_out(s):
            pltpu.make_async_copy(
                outs[s], o_hbm.at[pl.ds(base, _SC_CH)], sos[s]).wait()

        def compute(s):
            xin = ins[s]
            yout = outs[s]

            def vbody(o2):
                xv = xin[pl.ds(o2, 16)]
                node = jnp.ones((16,), jnp.int32)
                for _ in range(4):
                    tv = plsc.load_gather(tree_v, [node])
                    node = node + node + (xv >= tv).astype(jnp.int32)
                pk = plsc.load_gather(ptab_v, [node - 16])
                slope = plsc.bitcast(pk & _SLOPE_MASK, jnp.float32)
                b = (pk & _B_MASK).astype(jnp.float32) * (1.0 / _SCALE) - 1.0
                yout[pl.ds(o2, 16)] = b + slope * xv

            plsc.parallel_loop(0, _SC_CH, 16, unroll=8)(vbody)

        start_in(0, 0)
        start_in(1, 1)
        for g in (0, 1):
            wait_in(g)
            compute(g)
            start_in(g + 2, g)
            start_out(g, g)

        def pair_body(g2, c):
            g0 = 2 + 2 * g2
            for s in (0, 1):
                g = g0 + s
                wait_in(s)
                wait_out(s)
                compute(s)
                start_in(g + 2, s)
                start_out(g, s)
            return c

        lax.fori_loop(0, (nch - 4) // 2, pair_body, 0)

        for g in (nch - 2, nch - 1):
            s = g & 1
            wait_in(s)
            wait_out(s)
            compute(s)
            start_out(g, s)
        wait_out(0)
        wait_out(1)

    return sck(x, tree, ptab)


def kernel(x, breakpoints, intercepts, signs, exps):
    packed = _pack_tables(intercepts, signs, exps)
    thr = _thresholds(breakpoints)
    n = x.shape[0]
    cols = 1024
    k = _TC_ROWS_OF_16384 * (n // 16384)
    if k == 0:
        return _sc_call(x, _eytz_tree(thr), packed).astype(x.dtype)
    x2 = x.reshape(n // cols, cols)
    tc_out = _tc_call(x2, thr, packed, out_rows=k // cols)
    if k == n:
        return tc_out.reshape(n).astype(x.dtype)
    sc_out = _sc_call(x, _eytz_tree(thr), packed, skip=k)
    out = lax.dynamic_update_slice(sc_out, tc_out.reshape(-1), (0,))
    return out.astype(x.dtype)


# speedup vs baseline: 2.1447x; 1.9719x over previous
import functools

import jax
import jax.numpy as jnp
from jax import lax
from jax.experimental import pallas as pl
from jax.experimental.pallas import tpu as pltpu
from jax.experimental.pallas import tpu_sc as plsc

_SCALE = 65536.0
_NSEG = 16
_SLOPE_MASK = -8388608
_B_MASK = 0x1FFFF

_EYTZ_PERM = (8, 4, 9, 2, 10, 5, 11, 1, 12, 6, 13, 3, 14, 7, 15)

_TC_ELEMS = 10485760


def _pack_tables(intercepts, signs, exps):
    sign_bit = ((1 - signs) // 2).astype(jnp.int32)
    slope_bits = (sign_bit << 31) | ((127 + exps) << 23)
    return slope_bits | (intercepts + 65536)


def _thresholds(breakpoints):
    bpf = breakpoints.astype(jnp.float32)
    pos = bpf / _SCALE
    neg = jnp.nextafter(bpf - 1.0, jnp.float32(jnp.inf)) / _SCALE
    return jnp.where(breakpoints > 0, pos, neg)


def _eytz_tree(thresholds):
    return jnp.zeros((16,), jnp.float32).at[jnp.array(_EYTZ_PERM)].set(thresholds)


def _tc_body(t_ref, packed_ref, x_ref, o_ref):
    x = x_ref[...]
    acc = jnp.where(x >= t_ref[0], packed_ref[1], packed_ref[0])
    for j in range(1, _NSEG - 1):
        acc = jnp.where(x >= t_ref[j], packed_ref[j + 1], acc)
    slope = lax.bitcast_convert_type(acc & _SLOPE_MASK, jnp.float32)
    b = (acc & _B_MASK).astype(jnp.float32) * (1.0 / _SCALE) - 1.0
    o_ref[...] = b + slope * x


_TC_BLK = 524288


def _tc_call(x, thresholds, packed, out_elems=None):
    n = x.shape[0]
    out_elems = n if out_elems is None else out_elems
    grid = out_elems // _TC_BLK
    return pl.pallas_call(
        _tc_body,
        grid=(grid,),
        in_specs=[
            pl.BlockSpec(memory_space=pltpu.SMEM),
            pl.BlockSpec(memory_space=pltpu.SMEM),
            pl.BlockSpec((_TC_BLK,), lambda i: (i,)),
        ],
        out_specs=pl.BlockSpec((_TC_BLK,), lambda i: (i,)),
        out_shape=jax.ShapeDtypeStruct((out_elems,), jnp.float32),
    )(thresholds, packed, x)


_SC_CH = 16384


def _sc_call(x, tree, ptab, skip=0):
    n = x.shape[0]
    nw = 32
    per_w = (n - skip) // nw
    nch = per_w // _SC_CH
    assert (n - skip) % nw == 0 and per_w % _SC_CH == 0 and nch % 2 == 0
    mesh = plsc.VectorSubcoreMesh(core_axis_name="c", subcore_axis_name="s")

    @functools.partial(
        pl.kernel,
        out_type=jax.ShapeDtypeStruct((n - skip,), jnp.float32),
        mesh=mesh,
        compiler_params=pltpu.CompilerParams(needs_layout_passes=False),
        scratch_types=[
            pltpu.VMEM((16,), jnp.float32),
            pltpu.VMEM((16,), jnp.int32),
            pltpu.VMEM((_SC_CH,), jnp.float32),
            pltpu.VMEM((_SC_CH,), jnp.float32),
            pltpu.VMEM((_SC_CH,), jnp.float32),
            pltpu.VMEM((_SC_CH,), jnp.float32),
            pltpu.SemaphoreType.DMA,
            pltpu.SemaphoreType.DMA,
            pltpu.SemaphoreType.DMA,
            pltpu.SemaphoreType.DMA,
        ],
    )
    def sck(x_hbm, tree_hbm, ptab_hbm, o_hbm, tree_v, ptab_v,
            in0, in1, out0, out1, si0, si1, so0, so1):
        wid = lax.axis_index("s") * 2 + lax.axis_index("c")
        rbase = skip + wid * per_w
        base = wid * per_w
        pltpu.sync_copy(tree_hbm, tree_v)
        pltpu.sync_copy(ptab_hbm, ptab_v)
        ins = (in0, in1)
        outs = (out0, out1)
        sis = (si0, si1)
        sos = (so0, so1)

        def start_in(g, s):
            pltpu.async_copy(
                x_hbm.at[pl.ds(rbase + g * _SC_CH, _SC_CH)], ins[s], sis[s])

        def start_out(g, s):
            pltpu.async_copy(
                outs[s], o_hbm.at[pl.ds(base + g * _SC_CH, _SC_CH)], sos[s])

        def wait_in(s):
            pltpu.make_async_copy(
                x_hbm.at[pl.ds(rbase, _SC_CH)], ins[s], sis[s]).wait()

        def wait_out(s):
            pltpu.make_async_copy(
                outs[s], o_hbm.at[pl.ds(base, _SC_CH)], sos[s]).wait()

        def compute(s):
            xin = ins[s]
            yout = outs[s]

            def vbody(o2):
                xv = xin[pl.ds(o2, 16)]
                node = jnp.ones((16,), jnp.int32)
                for _ in range(4):
                    tv = plsc.load_gather(tree_v, [node])
                    node = node + node + (xv >= tv).astype(jnp.int32)
                pk = plsc.load_gather(ptab_v, [node - 16])
                slope = plsc.bitcast(pk & _SLOPE_MASK, jnp.float32)
                b = (pk & _B_MASK).astype(jnp.float32) * (1.0 / _SCALE) - 1.0
                yout[pl.ds(o2, 16)] = b + slope * xv

            plsc.parallel_loop(0, _SC_CH, 16, unroll=8)(vbody)

        start_in(0, 0)
        start_in(1, 1)
        for g in (0, 1):
            wait_in(g)
            compute(g)
            start_in(g + 2, g)
            start_out(g, g)

        def pair_body(g2, c):
            g0 = 2 + 2 * g2
            for s in (0, 1):
                g = g0 + s
                wait_in(s)
                wait_out(s)
                compute(s)
                start_in(g + 2, s)
                start_out(g, s)
            return c

        lax.fori_loop(0, (nch - 4) // 2, pair_body, 0)

        for g in (nch - 2, nch - 1):
            s = g & 1
            wait_in(s)
            wait_out(s)
            compute(s)
            start_out(g, s)
        wait_out(0)
        wait_out(1)

    return sck(x, tree, ptab)


def kernel(x, breakpoints, intercepts, signs, exps):
    packed = _pack_tables(intercepts, signs, exps)
    thr = _thresholds(breakpoints)
    n = x.shape[0]
    k = min(_TC_ELEMS, n) if n % _TC_BLK == 0 else 0
    if k == 0:
        return _sc_call(x, _eytz_tree(thr), packed).astype(x.dtype)
    if k == n:
        return _tc_call(x, thr, packed).astype(x.dtype)
    sc_out = _sc_call(x, _eytz_tree(thr), packed, skip=k)
    tc_full = pl.pallas_call(
        _tc_body,
        grid=(k // _TC_BLK,),
        in_specs=[
            pl.BlockSpec(memory_space=pltpu.SMEM),
            pl.BlockSpec(memory_space=pltpu.SMEM),
            pl.BlockSpec((_TC_BLK,), lambda i: (i,)),
        ],
        out_specs=pl.BlockSpec((_TC_BLK,), lambda i: (i,)),
        out_shape=jax.ShapeDtypeStruct((n,), jnp.float32),
    )(thr, packed, x)
    out = lax.dynamic_update_slice(tc_full, sc_out, (k,))
    return out.astype(x.dtype)


# speedup vs baseline: 2.8456x; 1.3268x over previous
import functools

import jax
import jax.numpy as jnp
from jax import lax
from jax.experimental import pallas as pl
from jax.experimental.pallas import tpu as pltpu
from jax.experimental.pallas import tpu_sc as plsc

_SCALE = 65536.0
_NSEG = 16
_SLOPE_MASK = -8388608
_B_MASK = 0x1FFFF

_EYTZ_PERM = (8, 4, 9, 2, 10, 5, 11, 1, 12, 6, 13, 3, 14, 7, 15)

_TC_ELEMS = 10485760


def _pack_tables(intercepts, signs, exps):
    sign_bit = ((1 - signs) // 2).astype(jnp.int32)
    slope_bits = (sign_bit << 31) | ((127 + exps) << 23)
    return slope_bits | (intercepts + 65536)


def _thresholds(breakpoints):
    bpf = breakpoints.astype(jnp.float32)
    pos = bpf / _SCALE
    neg = jnp.nextafter(bpf - 1.0, jnp.float32(jnp.inf)) / _SCALE
    return jnp.where(breakpoints > 0, pos, neg)


def _eytz_tree(thresholds):
    return jnp.zeros((16,), jnp.float32).at[jnp.array(_EYTZ_PERM)].set(thresholds)


def _tc_body(t_ref, packed_ref, x_ref, o_ref):
    x = x_ref[...]
    acc = jnp.where(x >= t_ref[0], packed_ref[1], packed_ref[0])
    for j in range(1, _NSEG - 1):
        acc = jnp.where(x >= t_ref[j], packed_ref[j + 1], acc)
    slope = lax.bitcast_convert_type(acc & _SLOPE_MASK, jnp.float32)
    b = (acc & _B_MASK).astype(jnp.float32) * (1.0 / _SCALE) - 1.0
    o_ref[...] = b + slope * x


_TC_BLK = 524288


def _tc_call(x, thresholds, packed, out_elems=None):
    n = x.shape[0]
    out_elems = n if out_elems is None else out_elems
    grid = out_elems // _TC_BLK
    return pl.pallas_call(
        _tc_body,
        grid=(grid,),
        in_specs=[
            pl.BlockSpec(memory_space=pltpu.SMEM),
            pl.BlockSpec(memory_space=pltpu.SMEM),
            pl.BlockSpec((_TC_BLK,), lambda i: (i,)),
        ],
        out_specs=pl.BlockSpec((_TC_BLK,), lambda i: (i,)),
        out_shape=jax.ShapeDtypeStruct((out_elems,), jnp.float32),
    )(thresholds, packed, x)


_SC_CH = 16384


def _sc_call(x, tree, ptab, skip=0):
    n = x.shape[0]
    nw = 32
    per_w = (n - skip) // nw
    nch = per_w // _SC_CH
    assert (n - skip) % nw == 0 and per_w % _SC_CH == 0 and nch % 2 == 0
    mesh = plsc.VectorSubcoreMesh(core_axis_name="c", subcore_axis_name="s")

    @functools.partial(
        pl.kernel,
        out_type=jax.ShapeDtypeStruct((n - skip,), jnp.float32),
        mesh=mesh,
        compiler_params=pltpu.CompilerParams(needs_layout_passes=False),
        scratch_types=[
            pltpu.VMEM((16,), jnp.float32),
            pltpu.VMEM((16,), jnp.int32),
            pltpu.VMEM((_SC_CH,), jnp.float32),
            pltpu.VMEM((_SC_CH,), jnp.float32),
            pltpu.VMEM((_SC_CH,), jnp.float32),
            pltpu.VMEM((_SC_CH,), jnp.float32),
            pltpu.SemaphoreType.DMA,
            pltpu.SemaphoreType.DMA,
            pltpu.SemaphoreType.DMA,
            pltpu.SemaphoreType.DMA,
        ],
    )
    def sck(x_hbm, tree_hbm, ptab_hbm, o_hbm, tree_v, ptab_v,
            in0, in1, out0, out1, si0, si1, so0, so1):
        wid = lax.axis_index("s") * 2 + lax.axis_index("c")
        rbase = skip + wid * per_w
        base = wid * per_w
        pltpu.sync_copy(tree_hbm, tree_v)
        pltpu.sync_copy(ptab_hbm, ptab_v)
        ins = (in0, in1)
        outs = (out0, out1)
        sis = (si0, si1)
        sos = (so0, so1)

        def start_in(g, s):
            pltpu.async_copy(
                x_hbm.at[pl.ds(rbase + g * _SC_CH, _SC_CH)], ins[s], sis[s])

        def start_out(g, s):
            pltpu.async_copy(
                outs[s], o_hbm.at[pl.ds(base + g * _SC_CH, _SC_CH)], sos[s])

        def wait_in(s):
            pltpu.make_async_copy(
                x_hbm.at[pl.ds(rbase, _SC_CH)], ins[s], sis[s]).wait()

        def wait_out(s):
            pltpu.make_async_copy(
                outs[s], o_hbm.at[pl.ds(base, _SC_CH)], sos[s]).wait()

        def compute(s):
            xin = ins[s]
            yout = outs[s]

            def vbody(o2):
                xv = xin[pl.ds(o2, 16)]
                node = jnp.ones((16,), jnp.int32)
                for _ in range(4):
                    tv = plsc.load_gather(tree_v, [node])
                    node = node + node + (xv >= tv).astype(jnp.int32)
                pk = plsc.load_gather(ptab_v, [node - 16])
                slope = plsc.bitcast(pk & _SLOPE_MASK, jnp.float32)
                b = (pk & _B_MASK).astype(jnp.float32) * (1.0 / _SCALE) - 1.0
                yout[pl.ds(o2, 16)] = b + slope * xv

            plsc.parallel_loop(0, _SC_CH, 16, unroll=8)(vbody)

        start_in(0, 0)
        start_in(1, 1)
        for g in (0, 1):
            wait_in(g)
            compute(g)
            start_in(g + 2, g)
            start_out(g, g)

        def pair_body(g2, c):
            g0 = 2 + 2 * g2
            for s in (0, 1):
                g = g0 + s
                wait_in(s)
                wait_out(s)
                compute(s)
                start_in(g + 2, s)
                start_out(g, s)
            return c

        lax.fori_loop(0, (nch - 4) // 2, pair_body, 0)

        for g in (nch - 2, nch - 1):
            s = g & 1
            wait_in(s)
            wait_out(s)
            compute(s)
            start_out(g, s)
        wait_out(0)
        wait_out(1)

    return sck(x, tree, ptab)


def kernel(x, breakpoints, intercepts, signs, exps):
    packed = _pack_tables(intercepts, signs, exps)
    thr = _thresholds(breakpoints)
    n = x.shape[0]
    k = min(_TC_ELEMS, n) if n % _TC_BLK == 0 else 0
    if k == 0:
        return _sc_call(x, _eytz_tree(thr), packed).astype(x.dtype)
    if k == n:
        return _tc_call(x, thr, packed).astype(x.dtype)
    sc_out = _sc_call(x, _eytz_tree(thr), packed, skip=k)
    cols = 128
    br = _TC_BLK // cols
    x2 = x.reshape(n // cols, cols)
    tc_full = pl.pallas_call(
        _tc_body,
        grid=(k // _TC_BLK,),
        in_specs=[
            pl.BlockSpec(memory_space=pltpu.SMEM),
            pl.BlockSpec(memory_space=pltpu.SMEM),
            pl.BlockSpec((br, cols), lambda i: (i, 0)),
        ],
        out_specs=pl.BlockSpec((br, cols), lambda i: (i, 0)),
        out_shape=jax.ShapeDtypeStruct((n // cols, cols), jnp.float32),
    )(thr, packed, x2)
    out = lax.dynamic_update_slice(
        tc_full, sc_out.reshape((n - k) // cols, cols), (k // cols, 0))
    return out.reshape(n).astype(x.dtype)
